# baseline (speedup 1.0000x reference)
import functools

import jax
import jax.numpy as jnp
from jax.experimental import pallas as pl
from jax.experimental.pallas import tpu as pltpu

N_EMBD = 4096
N_EXPERTS = 64
TOP_K = 8

_TOKEN_BLOCK = 512


def _router_block(x_ref, wt_ref, w_out_ref, i_out_ref, l_out_ref):
    logits = jax.lax.dot_general(
        x_ref[...], wt_ref[...],
        dimension_numbers=(((1,), (0,)), ((), ())),
        preferred_element_type=jnp.float32,
    )
    l_out_ref[...] = logits

    tb = logits.shape[0]
    lane = jax.lax.broadcasted_iota(jnp.int32, (tb, N_EXPERTS), 1)
    cur = logits
    vals = []
    idxs = []
    for _ in range(TOP_K):
        m = jnp.max(cur, axis=1, keepdims=True)
        sel = jnp.min(jnp.where(cur == m, lane, N_EXPERTS), axis=1, keepdims=True)
        vals.append(m)
        idxs.append(sel)
        cur = jnp.where(lane == sel, -jnp.inf, cur)
    top_vals = jnp.concatenate(vals, axis=1)
    top_idxs = jnp.concatenate(idxs, axis=1)

    e = jnp.exp(top_vals - top_vals[:, 0:1])
    w_out_ref[...] = e / jnp.sum(e, axis=1, keepdims=True)
    i_out_ref[...] = top_idxs


@functools.partial(jax.jit, static_argnames=("interpret",))
def kernel(x, W, interpret=False):
    b, t, c = x.shape
    n_tok = b * t
    xf = x.reshape(n_tok, c)
    wt = W.T

    grid = (n_tok // _TOKEN_BLOCK,)
    weights, indices, logits = pl.pallas_call(
        _router_block,
        grid=grid,
        in_specs=[
            pl.BlockSpec((_TOKEN_BLOCK, c), lambda i: (i, 0)),
            pl.BlockSpec((c, N_EXPERTS), lambda i: (0, 0)),
        ],
        out_specs=[
            pl.BlockSpec((_TOKEN_BLOCK, TOP_K), lambda i: (i, 0)),
            pl.BlockSpec((_TOKEN_BLOCK, TOP_K), lambda i: (i, 0)),
            pl.BlockSpec((_TOKEN_BLOCK, N_EXPERTS), lambda i: (i, 0)),
        ],
        out_shape=[
            jax.ShapeDtypeStruct((n_tok, TOP_K), jnp.float32),
            jax.ShapeDtypeStruct((n_tok, TOP_K), jnp.int32),
            jax.ShapeDtypeStruct((n_tok, N_EXPERTS), jnp.float32),
        ],
        interpret=interpret,
    )(xf, wt)

    return (weights.reshape(b, t, TOP_K),
            indices.reshape(b, t, TOP_K),
            logits.reshape(b, t, N_EXPERTS))


# speedup vs baseline: 1.1067x; 1.1067x over previous
import functools

import jax
import jax.numpy as jnp
from jax.experimental import pallas as pl
from jax.experimental.pallas import tpu as pltpu

N_EMBD = 4096
N_EXPERTS = 64
TOP_K = 8

_TOKEN_BLOCK = 512


def _router_block(x_ref, wt_ref, w_out_ref, i_out_ref, l_out_ref):
    logits = jax.lax.dot_general(
        x_ref[...], wt_ref[...],
        dimension_numbers=(((1,), (0,)), ((), ())),
        preferred_element_type=jnp.float32,
    )
    l_out_ref[...] = logits

    tb = logits.shape[0]
    lane_f = jax.lax.broadcasted_iota(
        jnp.int32, (tb, N_EXPERTS), 1).astype(jnp.float32)
    big = jnp.float32(N_EXPERTS)
    cur = logits
    vals = []
    idxs_f = []
    for _ in range(TOP_K):
        m = jnp.max(cur, axis=1, keepdims=True)
        sel = jnp.min(jnp.where(cur == m, lane_f, big), axis=1, keepdims=True)
        vals.append(m)
        idxs_f.append(sel)
        cur = jnp.where(lane_f == sel, -jnp.inf, cur)
    top_vals = jnp.concatenate(vals, axis=1)
    top_idxs = jnp.concatenate(idxs_f, axis=1).astype(jnp.int32)

    e = jnp.exp(top_vals - top_vals[:, 0:1])
    w_out_ref[...] = e / jnp.sum(e, axis=1, keepdims=True)
    i_out_ref[...] = top_idxs


@functools.partial(jax.jit, static_argnames=("interpret",))
def kernel(x, W, interpret=False):
    b, t, c = x.shape
    n_tok = b * t
    xf = x.reshape(n_tok, c)
    wt = W.T

    grid = (n_tok // _TOKEN_BLOCK,)
    weights, indices, logits = pl.pallas_call(
        _router_block,
        grid=grid,
        in_specs=[
            pl.BlockSpec((_TOKEN_BLOCK, c), lambda i: (i, 0)),
            pl.BlockSpec((c, N_EXPERTS), lambda i: (0, 0)),
        ],
        out_specs=[
            pl.BlockSpec((_TOKEN_BLOCK, TOP_K), lambda i: (i, 0)),
            pl.BlockSpec((_TOKEN_BLOCK, TOP_K), lambda i: (i, 0)),
            pl.BlockSpec((_TOKEN_BLOCK, N_EXPERTS), lambda i: (i, 0)),
        ],
        out_shape=[
            jax.ShapeDtypeStruct((n_tok, TOP_K), jnp.float32),
            jax.ShapeDtypeStruct((n_tok, TOP_K), jnp.int32),
            jax.ShapeDtypeStruct((n_tok, N_EXPERTS), jnp.float32),
        ],
        interpret=interpret,
    )(xf, wt)

    return (weights.reshape(b, t, TOP_K),
            indices.reshape(b, t, TOP_K),
            logits.reshape(b, t, N_EXPERTS))


# speedup vs baseline: 1.1716x; 1.0586x over previous
import functools

import jax
import jax.numpy as jnp
from jax.experimental import pallas as pl
from jax.experimental.pallas import tpu as pltpu

N_EMBD = 4096
N_EXPERTS = 64
TOP_K = 8

_TOKEN_BLOCK = 1024


def _router_block(x_ref, wt_ref, w_out_ref, i_out_ref, l_out_ref):
    logits = jax.lax.dot_general(
        x_ref[...], wt_ref[...],
        dimension_numbers=(((1,), (0,)), ((), ())),
        preferred_element_type=jnp.float32,
    )
    l_out_ref[...] = logits

    tb = logits.shape[0]
    lane_f = jax.lax.broadcasted_iota(
        jnp.int32, (tb, N_EXPERTS), 1).astype(jnp.float32)
    big = jnp.float32(N_EXPERTS)
    cur = logits
    vals = []
    idxs_f = []
    for _ in range(TOP_K):
        m = jnp.max(cur, axis=1, keepdims=True)
        sel = jnp.min(jnp.where(cur == m, lane_f, big), axis=1, keepdims=True)
        vals.append(m)
        idxs_f.append(sel)
        cur = jnp.where(lane_f == sel, -jnp.inf, cur)
    top_vals = jnp.concatenate(vals, axis=1)
    top_idxs = jnp.concatenate(idxs_f, axis=1).astype(jnp.int32)

    e = jnp.exp(top_vals - top_vals[:, 0:1])
    w_out_ref[...] = e / jnp.sum(e, axis=1, keepdims=True)
    i_out_ref[...] = top_idxs


@functools.partial(jax.jit, static_argnames=("interpret",))
def kernel(x, W, interpret=False):
    b, t, c = x.shape
    n_tok = b * t
    xf = x.reshape(n_tok, c)
    wt = W.T

    grid = (n_tok // _TOKEN_BLOCK,)
    weights, indices, logits = pl.pallas_call(
        _router_block,
        grid=grid,
        in_specs=[
            pl.BlockSpec((_TOKEN_BLOCK, c), lambda i: (i, 0)),
            pl.BlockSpec((c, N_EXPERTS), lambda i: (0, 0)),
        ],
        out_specs=[
            pl.BlockSpec((_TOKEN_BLOCK, TOP_K), lambda i: (i, 0)),
            pl.BlockSpec((_TOKEN_BLOCK, TOP_K), lambda i: (i, 0)),
            pl.BlockSpec((_TOKEN_BLOCK, N_EXPERTS), lambda i: (i, 0)),
        ],
        out_shape=[
            jax.ShapeDtypeStruct((n_tok, TOP_K), jnp.float32),
            jax.ShapeDtypeStruct((n_tok, TOP_K), jnp.int32),
            jax.ShapeDtypeStruct((n_tok, N_EXPERTS), jnp.float32),
        ],
        interpret=interpret,
    )(xf, wt)

    return (weights.reshape(b, t, TOP_K),
            indices.reshape(b, t, TOP_K),
            logits.reshape(b, t, N_EXPERTS))
